# baseline (speedup 1.0000x reference)
import functools

import jax
import jax.numpy as jnp
import numpy as np
from jax import lax
from jax.experimental import pallas as pl
from jax.experimental.pallas import tpu as pltpu
from jax.experimental.pallas import tpu_sc as plsc

BATCH = 4096
SEQ = 200
DIM = 64

NUM_CORES = 2
NUM_SUBCORES = 16
NUM_WORKERS = NUM_CORES * NUM_SUBCORES
ROWS_PER_WORKER = BATCH // NUM_WORKERS

G0, G1 = 112, 88
ACC_UNROLL = 4

WDIM = DIM // 2
_HI_MASK = np.int32(-65536)


def _sc_pool_body(ids_hbm, table_hbm, out_hbm,
                  ids_v, rows_a, rows_b, pooled_v, sem_a, sem_b):
  wid = lax.axis_index("s") * NUM_CORES + lax.axis_index("c")
  base = wid * ROWS_PER_WORKER

  pltpu.sync_copy(ids_hbm.at[pl.ds(base * SEQ, ROWS_PER_WORKER * SEQ)], ids_v)

  def remap_body(i, _):
    v = ids_v[pl.ds(i * 16, 16)]
    b = v >> _PB_LOG
    r = ((b >> 2) << (_PB_LOG + 2)) + ((v & (_PB - 1)) << 2) + (b & 3)
    ids_v[pl.ds(i * 16, 16)] = r
    return 0

  lax.fori_loop(0, ROWS_PER_WORKER * SEQ // 16, remap_body, 0)

  def issue(row, buf):
    idx0 = ids_v.at[pl.ds(row * SEQ, G0)]
    idx1 = ids_v.at[pl.ds(row * SEQ + G0, G1)]
    pltpu.async_copy(table_hbm.at[idx0], buf.at[pl.ds(0, G0)], _sem(buf))
    pltpu.async_copy(table_hbm.at[idx1], buf.at[pl.ds(G0, G1)], _sem(buf))

  def _sem(buf):
    return sem_a if buf is rows_a else sem_b

  def drain(buf):
    pltpu.make_async_copy(table_hbm.at[pl.ds(0, SEQ)], buf, _sem(buf)).wait()

  zerosf = jnp.zeros((16,), jnp.float32)

  def process(row, buf):
    def acc_body(i, carry):
      a0, a1, a2, a3 = carry
      for u in range(ACC_UNROLL):
        r = i * ACC_UNROLL + u
        v0 = buf[r, pl.ds(0, 16)]
        v1 = buf[r, pl.ds(16, 16)]
        a0 = a0 + plsc.bitcast(v0 << 16, jnp.float32)
        a1 = a1 + plsc.bitcast(v0 & _HI_MASK, jnp.float32)
        a2 = a2 + plsc.bitcast(v1 << 16, jnp.float32)
        a3 = a3 + plsc.bitcast(v1 & _HI_MASK, jnp.float32)
      return (a0, a1, a2, a3)

    a0, a1, a2, a3 = lax.fori_loop(
        0, SEQ // ACC_UNROLL, acc_body, (zerosf, zerosf, zerosf, zerosf))

    pooled_v[row, pl.ds(0, 16)] = a0
    pooled_v[row, pl.ds(16, 16)] = a1
    pooled_v[row, pl.ds(32, 16)] = a2
    pooled_v[row, pl.ds(48, 16)] = a3

  issue(0, rows_a)

  def outer(g2, _):
    g = g2 * 2
    issue(g + 1, rows_b)
    drain(rows_a)
    process(g, rows_a)

    @pl.when(g + 2 < ROWS_PER_WORKER)
    def _():
      issue(g + 2, rows_a)

    drain(rows_b)
    process(g + 1, rows_b)
    return 0

  lax.fori_loop(0, ROWS_PER_WORKER // 2, outer, 0)

  pltpu.sync_copy(pooled_v, out_hbm.at[pl.ds(base, ROWS_PER_WORKER)])


@functools.partial(jax.jit, static_argnames=())
def _sc_pool(input_ids, emb_table):
  mesh = plsc.VectorSubcoreMesh(
      core_axis_name="c", subcore_axis_name="s",
      num_cores=NUM_CORES, num_subcores=NUM_SUBCORES)
  f = pl.kernel(
      _sc_pool_body,
      out_type=jax.ShapeDtypeStruct((BATCH, DIM), jnp.float32),
      mesh=mesh,
      compiler_params=pltpu.CompilerParams(
          use_tc_tiling_on_sc=False, needs_layout_passes=False),
      scratch_types=[
          pltpu.VMEM((ROWS_PER_WORKER * SEQ,), jnp.int32),
          pltpu.VMEM((SEQ, WDIM), jnp.int32),
          pltpu.VMEM((SEQ, WDIM), jnp.int32),
          pltpu.VMEM((ROWS_PER_WORKER, DIM), jnp.float32),
          pltpu.SemaphoreType.DMA,
          pltpu.SemaphoreType.DMA,
      ],
  )
  return f(input_ids, emb_table)


_INV_SQRT2 = np.float32(1.0 / np.sqrt(2.0))

_PB = 8192
_PB_LOG = _PB.bit_length() - 1

def _transpose_body(ta, tb, tc, td, eye_ref, o_ref):
  eye = eye_ref[...]

  def pack(t_ref):
    y = jax.lax.dot_general(t_ref[...], eye, (((0,), (0,)), ((), ())),
                            preferred_element_type=jnp.float32)
    u = jax.lax.bitcast_convert_type(y, jnp.int32) + 0x8000
    s = (u >> 16) & 0xFFFF
    t = u & _HI_MASK
    return s[:, 0:WDIM] | t[:, WDIM:DIM]

  o_ref[:, pl.ds(0, WDIM)] = pack(ta)
  o_ref[:, pl.ds(WDIM, WDIM)] = pack(tb)
  o_ref[:, pl.ds(2 * WDIM, WDIM)] = pack(tc)
  o_ref[:, pl.ds(3 * WDIM, WDIM)] = pack(td)


def _relayout_table(table_t):
  vocab = table_t.shape[1]
  grid = (vocab + 4 * _PB - 1) // (4 * _PB)
  last_blk = (vocab + _PB - 1) // _PB - 1

  def imap(h):
    return lambda i: (0, jnp.minimum(4 * i + h, last_blk))

  packed = pl.pallas_call(
      _transpose_body,
      grid=(grid,),
      in_specs=[
          pl.BlockSpec((DIM, _PB), imap(0)),
          pl.BlockSpec((DIM, _PB), imap(1)),
          pl.BlockSpec((DIM, _PB), imap(2)),
          pl.BlockSpec((DIM, _PB), imap(3)),
          pl.BlockSpec((DIM, DIM), lambda i: (0, 0)),
      ],
      out_specs=pl.BlockSpec((_PB, 4 * WDIM), lambda i: (i, 0)),
      out_shape=jax.ShapeDtypeStruct((grid * _PB, 4 * WDIM), jnp.int32),
  )(table_t, table_t, table_t, table_t, jnp.eye(DIM, dtype=jnp.float32))
  return packed.reshape(grid * 4 * _PB, WDIM)


def _mlp_body(ids_ref, sums_ref, w1t_ref, b1_ref, w2t_ref, b2_ref, o_ref):
  cnt = jnp.sum((ids_ref[...] != 0).astype(jnp.float32), axis=1,
                keepdims=True)
  x = sums_ref[...] / jnp.maximum(cnt, 1.0)
  h = jnp.dot(x, w1t_ref[...], preferred_element_type=jnp.float32)
  h = h + b1_ref[...]
  h = 0.5 * h * (1.0 + lax.erf(h * _INV_SQRT2))
  o = jnp.dot(h, w2t_ref[...], preferred_element_type=jnp.float32)
  o_ref[...] = o + b2_ref[...]


def _mlp(ids, sums, w1t, b1_2d, w2t_pad, b2_2d):
  nblk = 8
  blk = BATCH // nblk
  return pl.pallas_call(
      _mlp_body,
      grid=(nblk,),
      in_specs=[
          pl.BlockSpec((blk, SEQ), lambda i: (i, 0)),
          pl.BlockSpec((blk, DIM), lambda i: (i, 0)),
          pl.BlockSpec((DIM, DIM), lambda i: (0, 0)),
          pl.BlockSpec((1, DIM), lambda i: (0, 0)),
          pl.BlockSpec((DIM, 128), lambda i: (0, 0)),
          pl.BlockSpec((1, 128), lambda i: (0, 0)),
      ],
      out_specs=pl.BlockSpec((blk, 128), lambda i: (i, 0)),
      out_shape=jax.ShapeDtypeStruct((BATCH, 128), jnp.float32),
  )(ids, sums, w1t, b1_2d, w2t_pad, b2_2d)


_POOL_PERM = np.concatenate([
    np.arange(0, 16), np.arange(32, 48),
    np.arange(16, 32), np.arange(48, 64)])


def kernel(input_ids, emb_table, W1, b1, W2, b2):
  ids = input_ids.astype(jnp.int32)
  table_lin = _relayout_table(emb_table.T)
  sums = _sc_pool(ids.reshape(BATCH * SEQ), table_lin)
  w1t = W1.T[_POOL_PERM]
  w2t_pad = jnp.pad(W2.T, ((0, 0), (0, 128 - W2.shape[0])))
  b2_pad = jnp.pad(b2, (0, 128 - b2.shape[0]))
  out = _mlp(ids, sums, w1t, b1.reshape(1, DIM), w2t_pad,
             b2_pad.reshape(1, 128))
  return out[:, :3]


# speedup vs baseline: 1.1960x; 1.1960x over previous
import functools

import jax
import jax.numpy as jnp
import numpy as np
from jax import lax
from jax.experimental import pallas as pl
from jax.experimental.pallas import tpu as pltpu
from jax.experimental.pallas import tpu_sc as plsc

BATCH = 4096
SEQ = 200
DIM = 64

NUM_CORES = 2
NUM_SUBCORES = 16
NUM_WORKERS = NUM_CORES * NUM_SUBCORES
ROWS_PER_WORKER = BATCH // NUM_WORKERS

G0, G1 = 112, 88
ACC_UNROLL = 4


def _sc_pool_body(ids_hbm, table_hbm, out_hbm,
                  ids_v, rows_a, rows_b, pooled_v, sem_a, sem_b):
  wid = lax.axis_index("s") * NUM_CORES + lax.axis_index("c")
  base = wid * ROWS_PER_WORKER

  pltpu.sync_copy(ids_hbm.at[pl.ds(base * SEQ, ROWS_PER_WORKER * SEQ)], ids_v)

  def remap_body(i, _):
    v = ids_v[pl.ds(i * 16, 16)]
    b = v >> _PB_LOG
    r = ((b >> 1) << (_PB_LOG + 1)) + ((v & (_PB - 1)) << 1) + (b & 1)
    ids_v[pl.ds(i * 16, 16)] = r
    return 0

  lax.fori_loop(0, ROWS_PER_WORKER * SEQ // 16, remap_body, 0)

  def issue(row, buf):
    idx0 = ids_v.at[pl.ds(row * SEQ, G0)]
    idx1 = ids_v.at[pl.ds(row * SEQ + G0, G1)]
    pltpu.async_copy(table_hbm.at[idx0], buf.at[pl.ds(0, G0)], _sem(buf))
    pltpu.async_copy(table_hbm.at[idx1], buf.at[pl.ds(G0, G1)], _sem(buf))

  def _sem(buf):
    return sem_a if buf is rows_a else sem_b

  def drain(buf):
    pltpu.make_async_copy(table_hbm.at[pl.ds(0, SEQ)], buf, _sem(buf)).wait()

  zerosf = jnp.zeros((16,), jnp.float32)

  def process(row, buf):
    def acc_body(i, carry):
      a0, a1, a2, a3 = carry
      for u in range(ACC_UNROLL):
        r = i * ACC_UNROLL + u
        a0 = a0 + buf[r, pl.ds(0, 16)]
        a1 = a1 + buf[r, pl.ds(16, 16)]
        a2 = a2 + buf[r, pl.ds(32, 16)]
        a3 = a3 + buf[r, pl.ds(48, 16)]
      return (a0, a1, a2, a3)

    a0, a1, a2, a3 = lax.fori_loop(
        0, SEQ // ACC_UNROLL, acc_body, (zerosf, zerosf, zerosf, zerosf))

    pooled_v[row, pl.ds(0, 16)] = a0
    pooled_v[row, pl.ds(16, 16)] = a1
    pooled_v[row, pl.ds(32, 16)] = a2
    pooled_v[row, pl.ds(48, 16)] = a3

  issue(0, rows_a)

  def outer(g2, _):
    g = g2 * 2
    issue(g + 1, rows_b)
    drain(rows_a)
    process(g, rows_a)

    @pl.when(g + 2 < ROWS_PER_WORKER)
    def _():
      issue(g + 2, rows_a)

    drain(rows_b)
    process(g + 1, rows_b)
    return 0

  lax.fori_loop(0, ROWS_PER_WORKER // 2, outer, 0)

  pltpu.sync_copy(pooled_v, out_hbm.at[pl.ds(base, ROWS_PER_WORKER)])


@functools.partial(jax.jit, static_argnames=())
def _sc_pool(input_ids, emb_table):
  mesh = plsc.VectorSubcoreMesh(
      core_axis_name="c", subcore_axis_name="s",
      num_cores=NUM_CORES, num_subcores=NUM_SUBCORES)
  f = pl.kernel(
      _sc_pool_body,
      out_type=jax.ShapeDtypeStruct((BATCH, DIM), jnp.float32),
      mesh=mesh,
      compiler_params=pltpu.CompilerParams(use_tc_tiling_on_sc=False),
      scratch_types=[
          pltpu.VMEM((ROWS_PER_WORKER * SEQ,), jnp.int32),
          pltpu.VMEM((SEQ, DIM), jnp.float32),
          pltpu.VMEM((SEQ, DIM), jnp.float32),
          pltpu.VMEM((ROWS_PER_WORKER, DIM), jnp.float32),
          pltpu.SemaphoreType.DMA,
          pltpu.SemaphoreType.DMA,
      ],
  )
  return f(input_ids, emb_table)


_INV_SQRT2 = np.float32(1.0 / np.sqrt(2.0))

_T_BLK = 16384


_PB = 16384
_PB_LOG = _PB.bit_length() - 1


def _transpose_body(ta_ref, tb_ref, eye_ref, o_ref):
  ya = jax.lax.dot_general(
      ta_ref[...], eye_ref[...], (((0,), (0,)), ((), ())),
      preferred_element_type=jnp.float32)
  yb = jax.lax.dot_general(
      tb_ref[...], eye_ref[...], (((0,), (0,)), ((), ())),
      preferred_element_type=jnp.float32)
  o_ref[...] = jnp.concatenate([ya, yb], axis=1)


def _relayout_table(table_t):
  vocab = table_t.shape[1]
  grid = (vocab + 2 * _PB - 1) // (2 * _PB)
  eye = jnp.eye(DIM, dtype=jnp.float32)
  last_blk = (vocab + _PB - 1) // _PB - 1
  packed = pl.pallas_call(
      _transpose_body,
      grid=(grid,),
      in_specs=[
          pl.BlockSpec((DIM, _PB), lambda i: (0, 2 * i)),
          pl.BlockSpec((DIM, _PB),
                       lambda i: (0, jnp.minimum(2 * i + 1, last_blk))),
          pl.BlockSpec((DIM, DIM), lambda i: (0, 0)),
      ],
      out_specs=pl.BlockSpec((_PB, 2 * DIM), lambda i: (i, 0)),
      out_shape=jax.ShapeDtypeStruct((grid * _PB, 2 * DIM), jnp.float32),
  )(table_t, table_t, eye)
  return packed.reshape(grid * 2 * _PB, DIM)


def _mlp_body(ids_ref, sums_ref, w1t_ref, b1_ref, w2t_ref, b2_ref, o_ref):
  cnt = jnp.sum((ids_ref[...] != 0).astype(jnp.float32), axis=1,
                keepdims=True)
  x = sums_ref[...] / jnp.maximum(cnt, 1.0)
  h = jnp.dot(x, w1t_ref[...], preferred_element_type=jnp.float32)
  h = h + b1_ref[...]
  h = 0.5 * h * (1.0 + lax.erf(h * _INV_SQRT2))
  o = jnp.dot(h, w2t_ref[...], preferred_element_type=jnp.float32)
  o_ref[...] = o + b2_ref[...]


def _mlp(ids, sums, w1t, b1_2d, w2t_pad, b2_2d):
  nblk = 8
  blk = BATCH // nblk
  return pl.pallas_call(
      _mlp_body,
      grid=(nblk,),
      in_specs=[
          pl.BlockSpec((blk, SEQ), lambda i: (i, 0)),
          pl.BlockSpec((blk, DIM), lambda i: (i, 0)),
          pl.BlockSpec((DIM, DIM), lambda i: (0, 0)),
          pl.BlockSpec((1, DIM), lambda i: (0, 0)),
          pl.BlockSpec((DIM, 128), lambda i: (0, 0)),
          pl.BlockSpec((1, 128), lambda i: (0, 0)),
      ],
      out_specs=pl.BlockSpec((blk, 128), lambda i: (i, 0)),
      out_shape=jax.ShapeDtypeStruct((BATCH, 128), jnp.float32),
  )(ids, sums, w1t, b1_2d, w2t_pad, b2_2d)


def kernel(input_ids, emb_table, W1, b1, W2, b2):
  ids = input_ids.astype(jnp.int32)
  table_lin = _relayout_table(emb_table.T)
  sums = _sc_pool(ids.reshape(BATCH * SEQ), table_lin)
  w1t = W1.T
  w2t_pad = jnp.pad(W2.T, ((0, 0), (0, 128 - W2.shape[0])))
  b2_pad = jnp.pad(b2, (0, 128 - b2.shape[0]))
  out = _mlp(ids, sums, w1t, b1.reshape(1, DIM), w2t_pad,
             b2_pad.reshape(1, 128))
  return out[:, :3]
